# Initial kernel scaffold; baseline (speedup 1.0000x reference)
import jax
import jax.numpy as jnp
from jax.experimental import pallas as pl


def kernel(x_user, x_item, ln1_g_user, ln1_b_user, ln1_g_item, ln1_b_item, W_self_user, W_self_item, W_u2i, W_i2u, ln2_g_user, ln2_b_user, ln2_g_item, ln2_b_item, W_mlp_user, b_mlp_user, W_mlp_item, b_mlp_item, edge_index_u2i, edge_index_i2u):
    raise NotImplementedError("write your pallas kernel here")


# speedup vs baseline: 3.1703x; 3.1703x over previous
import functools

import jax
import jax.numpy as jnp
from jax import lax
from jax.experimental import pallas as pl
from jax.experimental.pallas import tpu as pltpu
from jax.experimental.pallas import tpu_sc as plsc

N = 25000
D = 128
E = 300000
Q = 32
NS = 16
C = 128
K = -(-E // (NS * C))
EPAD = NS * C * K
NPAD = 25088
RPT = NPAD // NS
NB = 25
BR = N // NB


def _prep_body(x_ref, g_ref, b_ref, wrel_ref, wself_ref,
               t0_ref, t1_ref, t2_ref, t3_ref, s_ref):
    x = x_ref[...]
    mu = jnp.mean(x, axis=-1, keepdims=True)
    var = jnp.mean((x - mu) ** 2, axis=-1, keepdims=True)
    y = jnp.maximum((x - mu) * lax.rsqrt(var + 1e-5) * g_ref[0] + b_ref[0], 0.0)
    t = jnp.dot(y, wrel_ref[...], preferred_element_type=jnp.float32,
                precision=lax.Precision.HIGHEST)
    t0_ref[...] = t[:, 0 * Q:1 * Q]
    t1_ref[...] = t[:, 1 * Q:2 * Q]
    t2_ref[...] = t[:, 2 * Q:3 * Q]
    t3_ref[...] = t[:, 3 * Q:4 * Q]
    s_ref[...] = jnp.dot(y, wself_ref[...],
                         preferred_element_type=jnp.float32,
                         precision=lax.Precision.HIGHEST)


def _prep(x, g, b, wrel, wself):
    tq = jax.ShapeDtypeStruct((N, Q), jnp.float32)
    return pl.pallas_call(
        _prep_body,
        grid=(NB,),
        in_specs=[
            pl.BlockSpec((BR, D), lambda i: (i, 0)),
            pl.BlockSpec((1, D), lambda i: (0, 0)),
            pl.BlockSpec((1, D), lambda i: (0, 0)),
            pl.BlockSpec((D, D), lambda i: (0, 0)),
            pl.BlockSpec((D, D), lambda i: (0, 0)),
        ],
        out_specs=[
            pl.BlockSpec((BR, Q), lambda i: (i, 0)),
            pl.BlockSpec((BR, Q), lambda i: (i, 0)),
            pl.BlockSpec((BR, Q), lambda i: (i, 0)),
            pl.BlockSpec((BR, Q), lambda i: (i, 0)),
            pl.BlockSpec((BR, D), lambda i: (i, 0)),
        ],
        out_shape=[tq, tq, tq, tq,
                   jax.ShapeDtypeStruct((N, D), jnp.float32)],
    )(x, g, b, wrel, wself)


@functools.cache
def _build_sc_agg():
  mesh = plsc.VectorSubcoreMesh(core_axis_name="c", subcore_axis_name="s")

  @functools.partial(
    pl.kernel,
    out_type=[
        jax.ShapeDtypeStruct((4, NPAD, Q), jnp.float32),
        jax.ShapeDtypeStruct((NPAD, 16), jnp.float32),
        jax.ShapeDtypeStruct((4, NPAD, Q), jnp.float32),
        jax.ShapeDtypeStruct((NPAD, 16), jnp.float32),
    ],
    mesh=mesh,
    compiler_params=pltpu.CompilerParams(use_tc_tiling_on_sc=False),
    scratch_types=[
        pltpu.VMEM_SHARED((NPAD, Q), jnp.float32),
        pltpu.VMEM_SHARED((NPAD, 16), jnp.float32),
        pltpu.VMEM((K, C), jnp.int32),
        pltpu.VMEM((K, C), jnp.int32),
        pltpu.VMEM((C, Q), jnp.float32),
        pltpu.VMEM((C, Q), jnp.float32),
        pltpu.VMEM((C, 16), jnp.float32),
        pltpu.VMEM((C, 16), jnp.float32),
        pltpu.SemaphoreType.DMA,
    ],
  )
  def sc_agg(tu0_hbm, tu1_hbm, tu2_hbm, tu3_hbm,
             ti0_hbm, ti1_hbm, ti2_hbm, ti3_hbm,
             src0_hbm, dst0_hbm, src1_hbm, dst1_hbm,
             aggu_hbm, cntu_hbm, aggi_hbm, cnti_hbm,
             acc_sp, cnt_sp, srcv, dstv, rows, zrows, onesv, z16, sem):
    c = lax.axis_index("c")
    s = lax.axis_index("s")

    def _fill_wide(i, carry):
        zrows[i // 2, pl.ds((i % 2) * 16, 16)] = jnp.zeros((16,), jnp.float32)
        return carry

    lax.fori_loop(0, C * 2, _fill_wide, 0)

    def _fill_narrow(i, carry):
        z16[i, pl.ds(0, 16)] = jnp.zeros((16,), jnp.float32)
        onesv[i, pl.ds(0, 16)] = jnp.ones((16,), jnp.float32)
        return carry

    lax.fori_loop(0, C, _fill_narrow, 0)

    row0 = s * RPT
    nfull = RPT // C
    rem = RPT % C

    for r in range(2):
        tabs = ((tu0_hbm, tu1_hbm, tu2_hbm, tu3_hbm) if r == 0 else
                (ti0_hbm, ti1_hbm, ti2_hbm, ti3_hbm))
        src_hbm = src0_hbm if r == 0 else src1_hbm
        dst_hbm = dst0_hbm if r == 0 else dst1_hbm
        agg_hbm = aggi_hbm if r == 0 else aggu_hbm
        cnt_hbm = cnti_hbm if r == 0 else cntu_hbm

        pltpu.sync_copy(src_hbm.at[s], srcv)
        pltpu.sync_copy(dst_hbm.at[s], dstv)

        for p in range(2):
            for q in range(nfull):
                pltpu.sync_copy(zrows, acc_sp.at[pl.ds(row0 + q * C, C)])
            pltpu.sync_copy(zrows.at[pl.ds(0, rem)],
                            acc_sp.at[pl.ds(row0 + nfull * C, rem)])
            if p == 0:
                for q in range(nfull):
                    pltpu.sync_copy(z16, cnt_sp.at[pl.ds(row0 + q * C, C)])
                pltpu.sync_copy(z16.at[pl.ds(0, rem)],
                                cnt_sp.at[pl.ds(row0 + nfull * C, rem)])

            plsc.subcore_barrier()

            def _edges(table_ref, do_cnt):
                def body(j, carry):
                    pltpu.async_copy(table_ref.at[srcv.at[j]], rows,
                                     sem).wait()
                    pltpu.sync_copy(rows, acc_sp.at[dstv.at[j]], add=True)
                    if do_cnt:
                        pltpu.sync_copy(onesv, cnt_sp.at[dstv.at[j]],
                                        add=True)
                    return carry

                lax.fori_loop(0, K, body, 0)

            @pl.when(c == 0)
            def _(tab=tabs[p], dc=(r == 0 and p == 0)):
                _edges(tab, dc)

            @pl.when(c == 1)
            def _(tab=tabs[2 + p], dc=(r == 1 and p == 0)):
                _edges(tab, dc)

            plsc.subcore_barrier()

            @pl.when(c == 0)
            def _(agg=agg_hbm, cnt=cnt_hbm, qi=p, dc=(r == 0 and p == 0)):
                pltpu.sync_copy(acc_sp.at[pl.ds(row0, RPT)],
                                agg.at[qi, pl.ds(row0, RPT)])
                if dc:
                    pltpu.sync_copy(cnt_sp.at[pl.ds(row0, RPT)],
                                    cnt.at[pl.ds(row0, RPT)])

            @pl.when(c == 1)
            def _(agg=agg_hbm, cnt=cnt_hbm, qi=2 + p, dc=(r == 1 and p == 0)):
                pltpu.sync_copy(acc_sp.at[pl.ds(row0, RPT)],
                                agg.at[qi, pl.ds(row0, RPT)])
                if dc:
                    pltpu.sync_copy(cnt_sp.at[pl.ds(row0, RPT)],
                                    cnt.at[pl.ds(row0, RPT)])

  return sc_agg


def _sc_agg(*args):
    return _build_sc_agg()(*args)


def _post_body(x_ref, s_ref, agg_ref, cnt_ref, g_ref, b_ref, w_ref, bm_ref,
               o_ref):
    x = x_ref[...]
    agg = jnp.concatenate([agg_ref[0], agg_ref[1], agg_ref[2], agg_ref[3]],
                          axis=-1)
    cnt = jnp.maximum(cnt_ref[...][:, 0:1], 1.0)
    x2 = x + s_ref[...] + agg / cnt
    mu = jnp.mean(x2, axis=-1, keepdims=True)
    var = jnp.mean((x2 - mu) ** 2, axis=-1, keepdims=True)
    z = jnp.maximum((x2 - mu) * lax.rsqrt(var + 1e-5) * g_ref[0] + b_ref[0],
                    0.0)
    o_ref[...] = x2 + jnp.dot(z, w_ref[...],
                              preferred_element_type=jnp.float32,
                              precision=lax.Precision.HIGHEST) + bm_ref[0]


def _post(x, sv, agg, cnt, g, b, w, bm):
    return pl.pallas_call(
        _post_body,
        grid=(NB,),
        in_specs=[
            pl.BlockSpec((BR, D), lambda i: (i, 0)),
            pl.BlockSpec((BR, D), lambda i: (i, 0)),
            pl.BlockSpec((4, BR, Q), lambda i: (0, i, 0)),
            pl.BlockSpec((BR, 16), lambda i: (i, 0)),
            pl.BlockSpec((1, D), lambda i: (0, 0)),
            pl.BlockSpec((1, D), lambda i: (0, 0)),
            pl.BlockSpec((D, D), lambda i: (0, 0)),
            pl.BlockSpec((1, D), lambda i: (0, 0)),
        ],
        out_specs=pl.BlockSpec((BR, D), lambda i: (i, 0)),
        out_shape=jax.ShapeDtypeStruct((N, D), jnp.float32),
    )(x, sv, agg, cnt, g, b, w, bm)


def _pad_edges(ei):
    pad = EPAD - E
    src = jnp.concatenate([ei[0], jnp.zeros((pad,), jnp.int32)])
    dst = jnp.concatenate([ei[1], jnp.full((pad,), N, jnp.int32)])
    return src.reshape(NS, K, C), dst.reshape(NS, K, C)


def kernel(x_user, x_item, ln1_g_user, ln1_b_user, ln1_g_item, ln1_b_item,
           W_self_user, W_self_item, W_u2i, W_i2u,
           ln2_g_user, ln2_b_user, ln2_g_item, ln2_b_item,
           W_mlp_user, b_mlp_user, W_mlp_item, b_mlp_item,
           edge_index_u2i, edge_index_i2u):
    tu = _prep(x_user, ln1_g_user.reshape(1, D), ln1_b_user.reshape(1, D),
               W_u2i, W_self_user)
    ti = _prep(x_item, ln1_g_item.reshape(1, D), ln1_b_item.reshape(1, D),
               W_i2u, W_self_item)
    s_u, s_i = tu[4], ti[4]

    src0, dst0 = _pad_edges(edge_index_u2i)
    src1, dst1 = _pad_edges(edge_index_i2u)

    agg_u, cnt_u, agg_i, cnt_i = _sc_agg(tu[0], tu[1], tu[2], tu[3],
                                         ti[0], ti[1], ti[2], ti[3],
                                         src0, dst0, src1, dst1)

    out_u = _post(x_user, s_u, agg_u, cnt_u, ln2_g_user.reshape(1, D),
                  ln2_b_user.reshape(1, D), W_mlp_user,
                  b_mlp_user.reshape(1, D))
    out_i = _post(x_item, s_i, agg_i, cnt_i, ln2_g_item.reshape(1, D),
                  ln2_b_item.reshape(1, D), W_mlp_item,
                  b_mlp_item.reshape(1, D))
    return (out_u, out_i)


# speedup vs baseline: 3.5148x; 1.1087x over previous
import functools

import jax
import jax.numpy as jnp
from jax import lax
from jax.experimental import pallas as pl
from jax.experimental.pallas import tpu as pltpu
from jax.experimental.pallas import tpu_sc as plsc

N = 25000
D = 128
E = 300000
Q = 32
NS = 16
C = 128
K = -(-E // (NS * C))
EPAD = NS * C * K
NPAD = 25088
RPT = NPAD // NS
NB = 25
BR = N // NB


def _prep_body(x_ref, g_ref, b_ref, wrel_ref, wself_ref,
               t0_ref, t1_ref, t2_ref, t3_ref, s_ref):
    x = x_ref[...]
    mu = jnp.mean(x, axis=-1, keepdims=True)
    var = jnp.mean((x - mu) ** 2, axis=-1, keepdims=True)
    y = jnp.maximum((x - mu) * lax.rsqrt(var + 1e-5) * g_ref[0] + b_ref[0], 0.0)
    t = jnp.dot(y, wrel_ref[...], preferred_element_type=jnp.float32,
                precision=lax.Precision.HIGHEST)
    t0_ref[...] = t[:, 0 * Q:1 * Q]
    t1_ref[...] = t[:, 1 * Q:2 * Q]
    t2_ref[...] = t[:, 2 * Q:3 * Q]
    t3_ref[...] = t[:, 3 * Q:4 * Q]
    s_ref[...] = jnp.dot(y, wself_ref[...],
                         preferred_element_type=jnp.float32,
                         precision=lax.Precision.HIGHEST)


def _prep(x, g, b, wrel, wself):
    tq = jax.ShapeDtypeStruct((N, Q), jnp.float32)
    return pl.pallas_call(
        _prep_body,
        grid=(NB,),
        in_specs=[
            pl.BlockSpec((BR, D), lambda i: (i, 0)),
            pl.BlockSpec((1, D), lambda i: (0, 0)),
            pl.BlockSpec((1, D), lambda i: (0, 0)),
            pl.BlockSpec((D, D), lambda i: (0, 0)),
            pl.BlockSpec((D, D), lambda i: (0, 0)),
        ],
        out_specs=[
            pl.BlockSpec((BR, Q), lambda i: (i, 0)),
            pl.BlockSpec((BR, Q), lambda i: (i, 0)),
            pl.BlockSpec((BR, Q), lambda i: (i, 0)),
            pl.BlockSpec((BR, Q), lambda i: (i, 0)),
            pl.BlockSpec((BR, D), lambda i: (i, 0)),
        ],
        out_shape=[tq, tq, tq, tq,
                   jax.ShapeDtypeStruct((N, D), jnp.float32)],
    )(x, g, b, wrel, wself)


@functools.cache
def _build_sc_agg():
  mesh = plsc.VectorSubcoreMesh(core_axis_name="c", subcore_axis_name="s")

  @functools.partial(
    pl.kernel,
    out_type=[
        jax.ShapeDtypeStruct((4, NPAD, Q), jnp.float32),
        jax.ShapeDtypeStruct((NPAD, 16), jnp.float32),
        jax.ShapeDtypeStruct((4, NPAD, Q), jnp.float32),
        jax.ShapeDtypeStruct((NPAD, 16), jnp.float32),
    ],
    mesh=mesh,
    compiler_params=pltpu.CompilerParams(use_tc_tiling_on_sc=False),
    scratch_types=[
        pltpu.VMEM_SHARED((NPAD, Q), jnp.float32),
        pltpu.VMEM_SHARED((NPAD, 16), jnp.float32),
        pltpu.VMEM((K, C), jnp.int32),
        pltpu.VMEM((K, C), jnp.int32),
        pltpu.VMEM((2, C, Q), jnp.float32),
        pltpu.VMEM((C, Q), jnp.float32),
        pltpu.VMEM((C, 16), jnp.float32),
        pltpu.VMEM((C, 16), jnp.float32),
        pltpu.SemaphoreType.DMA,
    ],
  )
  def sc_agg(tu0_hbm, tu1_hbm, tu2_hbm, tu3_hbm,
             ti0_hbm, ti1_hbm, ti2_hbm, ti3_hbm,
             src0_hbm, dst0_hbm, src1_hbm, dst1_hbm,
             aggu_hbm, cntu_hbm, aggi_hbm, cnti_hbm,
             acc_sp, cnt_sp, srcv, dstv, rows, zrows, onesv, z16, sem):
    c = lax.axis_index("c")
    s = lax.axis_index("s")

    def _fill_wide(i, carry):
        zrows[i // 2, pl.ds((i % 2) * 16, 16)] = jnp.zeros((16,), jnp.float32)
        return carry

    lax.fori_loop(0, C * 2, _fill_wide, 0)

    def _fill_narrow(i, carry):
        z16[i, pl.ds(0, 16)] = jnp.zeros((16,), jnp.float32)
        onesv[i, pl.ds(0, 16)] = jnp.ones((16,), jnp.float32)
        return carry

    lax.fori_loop(0, C, _fill_narrow, 0)

    row0 = s * RPT
    nfull = RPT // C
    rem = RPT % C

    for r in range(2):
        tabs = ((tu0_hbm, tu1_hbm, tu2_hbm, tu3_hbm) if r == 0 else
                (ti0_hbm, ti1_hbm, ti2_hbm, ti3_hbm))
        src_hbm = src0_hbm if r == 0 else src1_hbm
        dst_hbm = dst0_hbm if r == 0 else dst1_hbm
        agg_hbm = aggi_hbm if r == 0 else aggu_hbm
        cnt_hbm = cnti_hbm if r == 0 else cntu_hbm

        pltpu.sync_copy(src_hbm.at[s], srcv)
        pltpu.sync_copy(dst_hbm.at[s], dstv)

        for p in range(2):
            for q in range(nfull):
                pltpu.sync_copy(zrows, acc_sp.at[pl.ds(row0 + q * C, C)])
            pltpu.sync_copy(zrows.at[pl.ds(0, rem)],
                            acc_sp.at[pl.ds(row0 + nfull * C, rem)])
            if p == 0:
                for q in range(nfull):
                    pltpu.sync_copy(z16, cnt_sp.at[pl.ds(row0 + q * C, C)])
                pltpu.sync_copy(z16.at[pl.ds(0, rem)],
                                cnt_sp.at[pl.ds(row0 + nfull * C, rem)])

            plsc.subcore_barrier()

            def _edges(table_ref, do_cnt):
                pltpu.async_copy(table_ref.at[srcv.at[0]], rows.at[0], sem)

                def body(j, carry):
                    b = lax.rem(j, 2)
                    pltpu.make_async_copy(table_ref.at[srcv.at[j]],
                                          rows.at[b], sem).wait()

                    @pl.when(j + 1 < K)
                    def _():
                        pltpu.async_copy(table_ref.at[srcv.at[j + 1]],
                                         rows.at[1 - b], sem)

                    pltpu.sync_copy(rows.at[b], acc_sp.at[dstv.at[j]],
                                    add=True)
                    if do_cnt:
                        pltpu.sync_copy(onesv, cnt_sp.at[dstv.at[j]],
                                        add=True)
                    return carry

                lax.fori_loop(0, K, body, 0)

            @pl.when(c == 0)
            def _(tab=tabs[p], dc=(r == 0 and p == 0)):
                _edges(tab, dc)

            @pl.when(c == 1)
            def _(tab=tabs[2 + p], dc=(r == 1 and p == 0)):
                _edges(tab, dc)

            plsc.subcore_barrier()

            @pl.when(c == 0)
            def _(agg=agg_hbm, cnt=cnt_hbm, qi=p, dc=(r == 0 and p == 0)):
                pltpu.sync_copy(acc_sp.at[pl.ds(row0, RPT)],
                                agg.at[qi, pl.ds(row0, RPT)])
                if dc:
                    pltpu.sync_copy(cnt_sp.at[pl.ds(row0, RPT)],
                                    cnt.at[pl.ds(row0, RPT)])

            @pl.when(c == 1)
            def _(agg=agg_hbm, cnt=cnt_hbm, qi=2 + p, dc=(r == 1 and p == 0)):
                pltpu.sync_copy(acc_sp.at[pl.ds(row0, RPT)],
                                agg.at[qi, pl.ds(row0, RPT)])
                if dc:
                    pltpu.sync_copy(cnt_sp.at[pl.ds(row0, RPT)],
                                    cnt.at[pl.ds(row0, RPT)])

  return sc_agg


def _sc_agg(*args):
    return _build_sc_agg()(*args)


def _post_body(x_ref, s_ref, agg_ref, cnt_ref, g_ref, b_ref, w_ref, bm_ref,
               o_ref):
    x = x_ref[...]
    agg = jnp.concatenate([agg_ref[0], agg_ref[1], agg_ref[2], agg_ref[3]],
                          axis=-1)
    cnt = jnp.maximum(cnt_ref[...][:, 0:1], 1.0)
    x2 = x + s_ref[...] + agg / cnt
    mu = jnp.mean(x2, axis=-1, keepdims=True)
    var = jnp.mean((x2 - mu) ** 2, axis=-1, keepdims=True)
    z = jnp.maximum((x2 - mu) * lax.rsqrt(var + 1e-5) * g_ref[0] + b_ref[0],
                    0.0)
    o_ref[...] = x2 + jnp.dot(z, w_ref[...],
                              preferred_element_type=jnp.float32,
                              precision=lax.Precision.HIGHEST) + bm_ref[0]


def _post(x, sv, agg, cnt, g, b, w, bm):
    return pl.pallas_call(
        _post_body,
        grid=(NB,),
        in_specs=[
            pl.BlockSpec((BR, D), lambda i: (i, 0)),
            pl.BlockSpec((BR, D), lambda i: (i, 0)),
            pl.BlockSpec((4, BR, Q), lambda i: (0, i, 0)),
            pl.BlockSpec((BR, 16), lambda i: (i, 0)),
            pl.BlockSpec((1, D), lambda i: (0, 0)),
            pl.BlockSpec((1, D), lambda i: (0, 0)),
            pl.BlockSpec((D, D), lambda i: (0, 0)),
            pl.BlockSpec((1, D), lambda i: (0, 0)),
        ],
        out_specs=pl.BlockSpec((BR, D), lambda i: (i, 0)),
        out_shape=jax.ShapeDtypeStruct((N, D), jnp.float32),
    )(x, sv, agg, cnt, g, b, w, bm)


def _pad_edges(ei):
    pad = EPAD - E
    src = jnp.concatenate([ei[0], jnp.zeros((pad,), jnp.int32)])
    dst = jnp.concatenate([ei[1], jnp.full((pad,), N, jnp.int32)])
    return src.reshape(NS, K, C), dst.reshape(NS, K, C)


def kernel(x_user, x_item, ln1_g_user, ln1_b_user, ln1_g_item, ln1_b_item,
           W_self_user, W_self_item, W_u2i, W_i2u,
           ln2_g_user, ln2_b_user, ln2_g_item, ln2_b_item,
           W_mlp_user, b_mlp_user, W_mlp_item, b_mlp_item,
           edge_index_u2i, edge_index_i2u):
    tu = _prep(x_user, ln1_g_user.reshape(1, D), ln1_b_user.reshape(1, D),
               W_u2i, W_self_user)
    ti = _prep(x_item, ln1_g_item.reshape(1, D), ln1_b_item.reshape(1, D),
               W_i2u, W_self_item)
    s_u, s_i = tu[4], ti[4]

    src0, dst0 = _pad_edges(edge_index_u2i)
    src1, dst1 = _pad_edges(edge_index_i2u)

    agg_u, cnt_u, agg_i, cnt_i = _sc_agg(tu[0], tu[1], tu[2], tu[3],
                                         ti[0], ti[1], ti[2], ti[3],
                                         src0, dst0, src1, dst1)

    out_u = _post(x_user, s_u, agg_u, cnt_u, ln2_g_user.reshape(1, D),
                  ln2_b_user.reshape(1, D), W_mlp_user,
                  b_mlp_user.reshape(1, D))
    out_i = _post(x_item, s_i, agg_i, cnt_i, ln2_g_item.reshape(1, D),
                  ln2_b_item.reshape(1, D), W_mlp_item,
                  b_mlp_item.reshape(1, D))
    return (out_u, out_i)


# speedup vs baseline: 4.2788x; 1.2174x over previous
import functools

import jax
import jax.numpy as jnp
from jax import lax
from jax.experimental import pallas as pl
from jax.experimental.pallas import tpu as pltpu
from jax.experimental.pallas import tpu_sc as plsc

N = 25000
D = 128
E = 300000
Q = 32
NS = 16
C = 128
K = -(-E // (NS * C))
EPAD = NS * C * K
NPAD = 25088
RPT = NPAD // NS
NB = 25
BR = N // NB


def _prep_body(x_ref, g_ref, b_ref, wrel_ref, wself_ref,
               t0_ref, t1_ref, t2_ref, t3_ref, s_ref):
    x = x_ref[...]
    mu = jnp.mean(x, axis=-1, keepdims=True)
    var = jnp.mean((x - mu) ** 2, axis=-1, keepdims=True)
    y = jnp.maximum((x - mu) * lax.rsqrt(var + 1e-5) * g_ref[0] + b_ref[0], 0.0)
    t = jnp.dot(y, wrel_ref[...], preferred_element_type=jnp.float32,
                precision=lax.Precision.HIGHEST)
    t0_ref[...] = t[:, 0 * Q:1 * Q]
    t1_ref[...] = t[:, 1 * Q:2 * Q]
    t2_ref[...] = t[:, 2 * Q:3 * Q]
    t3_ref[...] = t[:, 3 * Q:4 * Q]
    s_ref[...] = jnp.dot(y, wself_ref[...],
                         preferred_element_type=jnp.float32,
                         precision=lax.Precision.HIGHEST)


def _prep(x, g, b, wrel, wself):
    tq = jax.ShapeDtypeStruct((N, Q), jnp.float32)
    return pl.pallas_call(
        _prep_body,
        grid=(NB,),
        in_specs=[
            pl.BlockSpec((BR, D), lambda i: (i, 0)),
            pl.BlockSpec((1, D), lambda i: (0, 0)),
            pl.BlockSpec((1, D), lambda i: (0, 0)),
            pl.BlockSpec((D, D), lambda i: (0, 0)),
            pl.BlockSpec((D, D), lambda i: (0, 0)),
        ],
        out_specs=[
            pl.BlockSpec((BR, Q), lambda i: (i, 0)),
            pl.BlockSpec((BR, Q), lambda i: (i, 0)),
            pl.BlockSpec((BR, Q), lambda i: (i, 0)),
            pl.BlockSpec((BR, Q), lambda i: (i, 0)),
            pl.BlockSpec((BR, D), lambda i: (i, 0)),
        ],
        out_shape=[tq, tq, tq, tq,
                   jax.ShapeDtypeStruct((N, D), jnp.float32)],
    )(x, g, b, wrel, wself)


@functools.cache
def _build_sc_agg():
  mesh = plsc.VectorSubcoreMesh(core_axis_name="c", subcore_axis_name="s")

  NBUF = 6
  LOOK = 3

  @functools.partial(
    pl.kernel,
    out_type=[
        jax.ShapeDtypeStruct((4, NPAD, Q), jnp.float32),
        jax.ShapeDtypeStruct((NPAD, Q), jnp.float32),
        jax.ShapeDtypeStruct((4, NPAD, Q), jnp.float32),
        jax.ShapeDtypeStruct((NPAD, Q), jnp.float32),
    ],
    mesh=mesh,
    compiler_params=pltpu.CompilerParams(use_tc_tiling_on_sc=False),
    scratch_types=[
        pltpu.VMEM_SHARED((NPAD, Q), jnp.float32),
        pltpu.VMEM((K, C), jnp.int32),
        pltpu.VMEM((K, C), jnp.int32),
        pltpu.VMEM((NBUF, C, Q), jnp.float32),
        pltpu.VMEM((C, Q), jnp.float32),
        pltpu.SemaphoreType.DMA,
        pltpu.SemaphoreType.DMA,
    ],
  )
  def sc_agg(tu0_hbm, tu1_hbm, tu2_hbm, tu3_hbm,
             ti0_hbm, ti1_hbm, ti2_hbm, ti3_hbm,
             src0_hbm, dst0_hbm, src1_hbm, dst1_hbm,
             aggu_hbm, cntu_hbm, aggi_hbm, cnti_hbm,
             acc_sp, srcv, dstv, rows, onesv, sem_g, sem_s):
    c = lax.axis_index("c")
    s = lax.axis_index("s")

    def _fill_ones(i, carry):
        onesv[i // 2, pl.ds((i % 2) * 16, 16)] = jnp.ones((16,), jnp.float32)
        return carry

    lax.fori_loop(0, C * 2, _fill_ones, 0)

    row0 = s * RPT
    nfull = RPT // C
    rem = RPT % C

    def _zero_stripe():
        def _fill_zero(i, carry):
            rows[0, i // 2, pl.ds((i % 2) * 16, 16)] = jnp.zeros(
                (16,), jnp.float32)
            return carry

        lax.fori_loop(0, C * 2, _fill_zero, 0)
        for q in range(nfull):
            pltpu.sync_copy(rows.at[0], acc_sp.at[pl.ds(row0 + q * C, C)])
        pltpu.sync_copy(rows.at[0, pl.ds(0, rem)],
                        acc_sp.at[pl.ds(row0 + nfull * C, rem)])

    def _retire_scatter():
        pltpu.make_async_copy(rows.at[0], acc_sp.at[dstv.at[0]],
                              sem_s).wait()

    def _edges(table_ref):
        for b in range(LOOK):
            pltpu.async_copy(table_ref.at[srcv.at[b]], rows.at[b], sem_g)

        def body(j, carry):
            b = lax.rem(j, NBUF)
            pltpu.make_async_copy(table_ref.at[srcv.at[j]], rows.at[b],
                                  sem_g).wait()
            pltpu.async_copy(rows.at[b], acc_sp.at[dstv.at[j]], sem_s,
                             add=True)

            @pl.when(j >= NBUF - LOOK)
            def _():
                _retire_scatter()

            @pl.when(j + LOOK < K)
            def _():
                pltpu.async_copy(table_ref.at[srcv.at[j + LOOK]],
                                 rows.at[lax.rem(j + LOOK, NBUF)], sem_g)

            return carry

        lax.fori_loop(0, K, body, 0)
        for _ in range(NBUF - LOOK):
            _retire_scatter()

    def _cnt_pass(cnt_hbm):
        _zero_stripe()
        plsc.subcore_barrier()

        def body(j, carry):
            pltpu.async_copy(onesv, acc_sp.at[dstv.at[j]], sem_s, add=True)

            @pl.when(j >= 8)
            def _():
                _retire_scatter()

            return carry

        lax.fori_loop(0, K, body, 0)
        for _ in range(8):
            _retire_scatter()
        plsc.subcore_barrier()
        pltpu.sync_copy(acc_sp.at[pl.ds(row0, RPT)],
                        cnt_hbm.at[pl.ds(row0, RPT)])

    for r in range(2):
        tabs = ((tu0_hbm, tu1_hbm, tu2_hbm, tu3_hbm) if r == 0 else
                (ti0_hbm, ti1_hbm, ti2_hbm, ti3_hbm))
        src_hbm = src0_hbm if r == 0 else src1_hbm
        dst_hbm = dst0_hbm if r == 0 else dst1_hbm
        agg_hbm = aggi_hbm if r == 0 else aggu_hbm
        cnt_hbm = cnti_hbm if r == 0 else cntu_hbm

        pltpu.sync_copy(src_hbm.at[s], srcv)
        pltpu.sync_copy(dst_hbm.at[s], dstv)

        for p in range(2):
            _zero_stripe()
            plsc.subcore_barrier()

            @pl.when(c == 0)
            def _(tab=tabs[p]):
                _edges(tab)

            @pl.when(c == 1)
            def _(tab=tabs[2 + p]):
                _edges(tab)

            plsc.subcore_barrier()

            @pl.when(c == 0)
            def _(agg=agg_hbm, qi=p):
                pltpu.sync_copy(acc_sp.at[pl.ds(row0, RPT)],
                                agg.at[qi, pl.ds(row0, RPT)])

            @pl.when(c == 1)
            def _(agg=agg_hbm, qi=2 + p):
                pltpu.sync_copy(acc_sp.at[pl.ds(row0, RPT)],
                                agg.at[qi, pl.ds(row0, RPT)])

        @pl.when(c == r)
        def _(cnt=cnt_hbm):
            _cnt_pass(cnt)

  return sc_agg


def _sc_agg(*args):
    return _build_sc_agg()(*args)


def _post_body(x_ref, s_ref, agg_ref, cnt_ref, g_ref, b_ref, w_ref, bm_ref,
               o_ref):
    x = x_ref[...]
    agg = jnp.concatenate([agg_ref[0], agg_ref[1], agg_ref[2], agg_ref[3]],
                          axis=-1)
    cnt = jnp.maximum(cnt_ref[...][:, 0:1], 1.0)
    x2 = x + s_ref[...] + agg / cnt
    mu = jnp.mean(x2, axis=-1, keepdims=True)
    var = jnp.mean((x2 - mu) ** 2, axis=-1, keepdims=True)
    z = jnp.maximum((x2 - mu) * lax.rsqrt(var + 1e-5) * g_ref[0] + b_ref[0],
                    0.0)
    o_ref[...] = x2 + jnp.dot(z, w_ref[...],
                              preferred_element_type=jnp.float32,
                              precision=lax.Precision.HIGHEST) + bm_ref[0]


def _post(x, sv, agg, cnt, g, b, w, bm):
    return pl.pallas_call(
        _post_body,
        grid=(NB,),
        in_specs=[
            pl.BlockSpec((BR, D), lambda i: (i, 0)),
            pl.BlockSpec((BR, D), lambda i: (i, 0)),
            pl.BlockSpec((4, BR, Q), lambda i: (0, i, 0)),
            pl.BlockSpec((BR, Q), lambda i: (i, 0)),
            pl.BlockSpec((1, D), lambda i: (0, 0)),
            pl.BlockSpec((1, D), lambda i: (0, 0)),
            pl.BlockSpec((D, D), lambda i: (0, 0)),
            pl.BlockSpec((1, D), lambda i: (0, 0)),
        ],
        out_specs=pl.BlockSpec((BR, D), lambda i: (i, 0)),
        out_shape=jax.ShapeDtypeStruct((N, D), jnp.float32),
    )(x, sv, agg, cnt, g, b, w, bm)


def _pad_edges(ei):
    pad = EPAD - E
    src = jnp.concatenate([ei[0], jnp.zeros((pad,), jnp.int32)])
    dst = jnp.concatenate([ei[1], jnp.full((pad,), N, jnp.int32)])
    return src.reshape(NS, K, C), dst.reshape(NS, K, C)


def kernel(x_user, x_item, ln1_g_user, ln1_b_user, ln1_g_item, ln1_b_item,
           W_self_user, W_self_item, W_u2i, W_i2u,
           ln2_g_user, ln2_b_user, ln2_g_item, ln2_b_item,
           W_mlp_user, b_mlp_user, W_mlp_item, b_mlp_item,
           edge_index_u2i, edge_index_i2u):
    tu = _prep(x_user, ln1_g_user.reshape(1, D), ln1_b_user.reshape(1, D),
               W_u2i, W_self_user)
    ti = _prep(x_item, ln1_g_item.reshape(1, D), ln1_b_item.reshape(1, D),
               W_i2u, W_self_item)
    s_u, s_i = tu[4], ti[4]

    src0, dst0 = _pad_edges(edge_index_u2i)
    src1, dst1 = _pad_edges(edge_index_i2u)

    agg_u, cnt_u, agg_i, cnt_i = _sc_agg(tu[0], tu[1], tu[2], tu[3],
                                         ti[0], ti[1], ti[2], ti[3],
                                         src0, dst0, src1, dst1)

    out_u = _post(x_user, s_u, agg_u, cnt_u, ln2_g_user.reshape(1, D),
                  ln2_b_user.reshape(1, D), W_mlp_user,
                  b_mlp_user.reshape(1, D))
    out_i = _post(x_item, s_i, agg_i, cnt_i, ln2_g_item.reshape(1, D),
                  ln2_b_item.reshape(1, D), W_mlp_item,
                  b_mlp_item.reshape(1, D))
    return (out_u, out_i)


# speedup vs baseline: 4.3994x; 1.0282x over previous
import functools

import jax
import jax.numpy as jnp
from jax import lax
from jax.experimental import pallas as pl
from jax.experimental.pallas import tpu as pltpu
from jax.experimental.pallas import tpu_sc as plsc

N = 25000
D = 128
E = 300000
Q = 32
NS = 16
C = 128
K = -(-E // (NS * C))
EPAD = NS * C * K
NPAD = 25088
RPT = NPAD // NS
NB = 25
BR = N // NB


def _prep_body(x_ref, g_ref, b_ref, wrel_ref, wself_ref,
               t0_ref, t1_ref, t2_ref, t3_ref, s_ref):
    x = x_ref[...]
    mu = jnp.mean(x, axis=-1, keepdims=True)
    var = jnp.mean((x - mu) ** 2, axis=-1, keepdims=True)
    y = jnp.maximum((x - mu) * lax.rsqrt(var + 1e-5) * g_ref[0] + b_ref[0], 0.0)
    t = jnp.dot(y, wrel_ref[...], preferred_element_type=jnp.float32,
                precision=lax.Precision.HIGHEST)
    t0_ref[...] = t[:, 0 * Q:1 * Q]
    t1_ref[...] = t[:, 1 * Q:2 * Q]
    t2_ref[...] = t[:, 2 * Q:3 * Q]
    t3_ref[...] = t[:, 3 * Q:4 * Q]
    s_ref[...] = jnp.dot(y, wself_ref[...],
                         preferred_element_type=jnp.float32,
                         precision=lax.Precision.HIGHEST)


def _prep(x, g, b, wrel, wself):
    tq = jax.ShapeDtypeStruct((N, Q), jnp.float32)
    return pl.pallas_call(
        _prep_body,
        grid=(NB,),
        in_specs=[
            pl.BlockSpec((BR, D), lambda i: (i, 0)),
            pl.BlockSpec((1, D), lambda i: (0, 0)),
            pl.BlockSpec((1, D), lambda i: (0, 0)),
            pl.BlockSpec((D, D), lambda i: (0, 0)),
            pl.BlockSpec((D, D), lambda i: (0, 0)),
        ],
        out_specs=[
            pl.BlockSpec((BR, Q), lambda i: (i, 0)),
            pl.BlockSpec((BR, Q), lambda i: (i, 0)),
            pl.BlockSpec((BR, Q), lambda i: (i, 0)),
            pl.BlockSpec((BR, Q), lambda i: (i, 0)),
            pl.BlockSpec((BR, D), lambda i: (i, 0)),
        ],
        out_shape=[tq, tq, tq, tq,
                   jax.ShapeDtypeStruct((N, D), jnp.float32)],
    )(x, g, b, wrel, wself)


@functools.cache
def _build_sc_agg():
  mesh = plsc.VectorSubcoreMesh(core_axis_name="c", subcore_axis_name="s")

  NBUF = 8
  LOOK = 4

  @functools.partial(
    pl.kernel,
    out_type=[
        jax.ShapeDtypeStruct((4, NPAD, Q), jnp.float32),
        jax.ShapeDtypeStruct((NPAD, Q), jnp.float32),
        jax.ShapeDtypeStruct((4, NPAD, Q), jnp.float32),
        jax.ShapeDtypeStruct((NPAD, Q), jnp.float32),
    ],
    mesh=mesh,
    compiler_params=pltpu.CompilerParams(use_tc_tiling_on_sc=False),
    scratch_types=[
        pltpu.VMEM_SHARED((NPAD, Q), jnp.float32),
        pltpu.VMEM((K, C), jnp.int32),
        pltpu.VMEM((K, C), jnp.int32),
        pltpu.VMEM((NBUF, C, Q), jnp.float32),
        pltpu.VMEM((C, Q), jnp.float32),
        pltpu.SemaphoreType.DMA,
        pltpu.SemaphoreType.DMA,
    ],
  )
  def sc_agg(tu0_hbm, tu1_hbm, tu2_hbm, tu3_hbm,
             ti0_hbm, ti1_hbm, ti2_hbm, ti3_hbm,
             src0_hbm, dst0_hbm, src1_hbm, dst1_hbm,
             aggu_hbm, cntu_hbm, aggi_hbm, cnti_hbm,
             acc_sp, srcv, dstv, rows, onesv, sem_g, sem_s):
    c = lax.axis_index("c")
    s = lax.axis_index("s")

    def _fill_ones(i, carry):
        onesv[i // 2, pl.ds((i % 2) * 16, 16)] = jnp.ones((16,), jnp.float32)
        return carry

    lax.fori_loop(0, C * 2, _fill_ones, 0)

    row0 = s * RPT
    nfull = RPT // C
    rem = RPT % C

    def _zero_stripe():
        def _fill_zero(i, carry):
            rows[0, i // 2, pl.ds((i % 2) * 16, 16)] = jnp.zeros(
                (16,), jnp.float32)
            return carry

        lax.fori_loop(0, C * 2, _fill_zero, 0)
        for q in range(nfull):
            pltpu.sync_copy(rows.at[0], acc_sp.at[pl.ds(row0 + q * C, C)])
        pltpu.sync_copy(rows.at[0, pl.ds(0, rem)],
                        acc_sp.at[pl.ds(row0 + nfull * C, rem)])

    def _retire_scatter():
        pltpu.make_async_copy(rows.at[0], acc_sp.at[dstv.at[0]],
                              sem_s).wait()

    def _edges(table_ref):
        for b in range(LOOK):
            pltpu.async_copy(table_ref.at[srcv.at[b]], rows.at[b], sem_g)

        def body(j, carry):
            b = lax.rem(j, NBUF)
            pltpu.make_async_copy(table_ref.at[srcv.at[j]], rows.at[b],
                                  sem_g).wait()
            pltpu.async_copy(rows.at[b], acc_sp.at[dstv.at[j]], sem_s,
                             add=True)

            @pl.when(j >= NBUF - LOOK)
            def _():
                _retire_scatter()

            @pl.when(j + LOOK < K)
            def _():
                pltpu.async_copy(table_ref.at[srcv.at[j + LOOK]],
                                 rows.at[lax.rem(j + LOOK, NBUF)], sem_g)

            return carry

        lax.fori_loop(0, K, body, 0)
        for _ in range(NBUF - LOOK):
            _retire_scatter()

    def _cnt_pass(cnt_hbm):
        _zero_stripe()
        plsc.subcore_barrier()

        def body(j, carry):
            pltpu.async_copy(onesv, acc_sp.at[dstv.at[j]], sem_s, add=True)

            @pl.when(j >= 8)
            def _():
                _retire_scatter()

            return carry

        lax.fori_loop(0, K, body, 0)
        for _ in range(8):
            _retire_scatter()
        plsc.subcore_barrier()
        pltpu.sync_copy(acc_sp.at[pl.ds(row0, RPT)],
                        cnt_hbm.at[pl.ds(row0, RPT)])

    for r in range(2):
        tabs = ((tu0_hbm, tu1_hbm, tu2_hbm, tu3_hbm) if r == 0 else
                (ti0_hbm, ti1_hbm, ti2_hbm, ti3_hbm))
        src_hbm = src0_hbm if r == 0 else src1_hbm
        dst_hbm = dst0_hbm if r == 0 else dst1_hbm
        agg_hbm = aggi_hbm if r == 0 else aggu_hbm
        cnt_hbm = cnti_hbm if r == 0 else cntu_hbm

        pltpu.sync_copy(src_hbm.at[s], srcv)
        pltpu.sync_copy(dst_hbm.at[s], dstv)

        for p in range(2):
            _zero_stripe()
            plsc.subcore_barrier()

            @pl.when(c == 0)
            def _(tab=tabs[p]):
                _edges(tab)

            @pl.when(c == 1)
            def _(tab=tabs[2 + p]):
                _edges(tab)

            plsc.subcore_barrier()

            @pl.when(c == 0)
            def _(agg=agg_hbm, qi=p):
                pltpu.sync_copy(acc_sp.at[pl.ds(row0, RPT)],
                                agg.at[qi, pl.ds(row0, RPT)])

            @pl.when(c == 1)
            def _(agg=agg_hbm, qi=2 + p):
                pltpu.sync_copy(acc_sp.at[pl.ds(row0, RPT)],
                                agg.at[qi, pl.ds(row0, RPT)])

        @pl.when(c == r)
        def _(cnt=cnt_hbm):
            _cnt_pass(cnt)

  return sc_agg


def _sc_agg(*args):
    return _build_sc_agg()(*args)


def _post_body(x_ref, s_ref, agg_ref, cnt_ref, g_ref, b_ref, w_ref, bm_ref,
               o_ref):
    x = x_ref[...]
    agg = jnp.concatenate([agg_ref[0], agg_ref[1], agg_ref[2], agg_ref[3]],
                          axis=-1)
    cnt = jnp.maximum(cnt_ref[...][:, 0:1], 1.0)
    x2 = x + s_ref[...] + agg / cnt
    mu = jnp.mean(x2, axis=-1, keepdims=True)
    var = jnp.mean((x2 - mu) ** 2, axis=-1, keepdims=True)
    z = jnp.maximum((x2 - mu) * lax.rsqrt(var + 1e-5) * g_ref[0] + b_ref[0],
                    0.0)
    o_ref[...] = x2 + jnp.dot(z, w_ref[...],
                              preferred_element_type=jnp.float32,
                              precision=lax.Precision.HIGHEST) + bm_ref[0]


def _post(x, sv, agg, cnt, g, b, w, bm):
    return pl.pallas_call(
        _post_body,
        grid=(NB,),
        in_specs=[
            pl.BlockSpec((BR, D), lambda i: (i, 0)),
            pl.BlockSpec((BR, D), lambda i: (i, 0)),
            pl.BlockSpec((4, BR, Q), lambda i: (0, i, 0)),
            pl.BlockSpec((BR, Q), lambda i: (i, 0)),
            pl.BlockSpec((1, D), lambda i: (0, 0)),
            pl.BlockSpec((1, D), lambda i: (0, 0)),
            pl.BlockSpec((D, D), lambda i: (0, 0)),
            pl.BlockSpec((1, D), lambda i: (0, 0)),
        ],
        out_specs=pl.BlockSpec((BR, D), lambda i: (i, 0)),
        out_shape=jax.ShapeDtypeStruct((N, D), jnp.float32),
    )(x, sv, agg, cnt, g, b, w, bm)


def _pad_edges(ei):
    pad = EPAD - E
    src = jnp.concatenate([ei[0], jnp.zeros((pad,), jnp.int32)])
    dst = jnp.concatenate([ei[1], jnp.full((pad,), N, jnp.int32)])
    return src.reshape(NS, K, C), dst.reshape(NS, K, C)


def kernel(x_user, x_item, ln1_g_user, ln1_b_user, ln1_g_item, ln1_b_item,
           W_self_user, W_self_item, W_u2i, W_i2u,
           ln2_g_user, ln2_b_user, ln2_g_item, ln2_b_item,
           W_mlp_user, b_mlp_user, W_mlp_item, b_mlp_item,
           edge_index_u2i, edge_index_i2u):
    tu = _prep(x_user, ln1_g_user.reshape(1, D), ln1_b_user.reshape(1, D),
               W_u2i, W_self_user)
    ti = _prep(x_item, ln1_g_item.reshape(1, D), ln1_b_item.reshape(1, D),
               W_i2u, W_self_item)
    s_u, s_i = tu[4], ti[4]

    src0, dst0 = _pad_edges(edge_index_u2i)
    src1, dst1 = _pad_edges(edge_index_i2u)

    agg_u, cnt_u, agg_i, cnt_i = _sc_agg(tu[0], tu[1], tu[2], tu[3],
                                         ti[0], ti[1], ti[2], ti[3],
                                         src0, dst0, src1, dst1)

    out_u = _post(x_user, s_u, agg_u, cnt_u, ln2_g_user.reshape(1, D),
                  ln2_b_user.reshape(1, D), W_mlp_user,
                  b_mlp_user.reshape(1, D))
    out_i = _post(x_item, s_i, agg_i, cnt_i, ln2_g_item.reshape(1, D),
                  ln2_b_item.reshape(1, D), W_mlp_item,
                  b_mlp_item.reshape(1, D))
    return (out_u, out_i)
